# Initial kernel scaffold; baseline (speedup 1.0000x reference)
import functools

import jax
import jax.numpy as jnp
from jax import lax
from jax.experimental import pallas as pl
from jax.experimental.pallas import tpu as pltpu
from jax.experimental.pallas import tpu_sc as plsc

BLK = 256
_NC, _NS = 2, 16
_NW = _NC * _NS


def _gating_body(x_ref, wg_ref, i1_ref, i2_ref, g1_ref, g2_ref, imp_ref,
                 load_ref):
    x = x_ref[...]
    wg = wg_ref[...]
    logits = jnp.dot(x, wg, preferred_element_type=jnp.float32)
    T, E = logits.shape
    iota_e = lax.broadcasted_iota(jnp.int32, (T, E), 1)
    m1 = jnp.max(logits, axis=1, keepdims=True)
    i1 = jnp.min(jnp.where(logits == m1, iota_e, E), axis=1, keepdims=True)
    masked = jnp.where(iota_e == i1, -jnp.inf, logits)
    m2 = jnp.max(masked, axis=1, keepdims=True)
    i2 = jnp.min(jnp.where(masked == m2, iota_e, E), axis=1, keepdims=True)
    e2 = jnp.exp(m2 - m1)
    den = 1.0 + e2
    g1 = 1.0 / den
    g2 = e2 / den
    i1_ref[...] = i1
    i2_ref[...] = i2
    g1_ref[...] = g1
    g2_ref[...] = g2
    oh1 = (iota_e == i1).astype(jnp.float32)
    oh2 = (iota_e == i2).astype(jnp.float32)
    imp_ref[...] = jnp.sum(oh1 * g1 + oh2 * g2, axis=0, keepdims=True)
    ld1 = jnp.where((iota_e == i1) & (g1 > 0), 1.0, 0.0)
    ld2 = jnp.where((iota_e == i2) & (g2 > 0), 1.0, 0.0)
    load_ref[...] = jnp.sum(ld1 + ld2, axis=0, keepdims=True)


def _gating(x, w_gate):
    T = x.shape[0]
    E = w_gate.shape[1]
    return pl.pallas_call(
        _gating_body,
        out_shape=[
            jax.ShapeDtypeStruct((T, 1), jnp.int32),
            jax.ShapeDtypeStruct((T, 1), jnp.int32),
            jax.ShapeDtypeStruct((T, 1), jnp.float32),
            jax.ShapeDtypeStruct((T, 1), jnp.float32),
            jax.ShapeDtypeStruct((1, E), jnp.float32),
            jax.ShapeDtypeStruct((1, E), jnp.float32),
        ],
    )(x, w_gate)


def _dispatch(x, sorted_tok, Rp):
    D = x.shape[1]
    rows_w = Rp // _NW
    CH = 64
    nch = rows_w // CH
    mesh = plsc.VectorSubcoreMesh(core_axis_name="c", subcore_axis_name="s")

    @functools.partial(
        pl.kernel,
        out_type=jax.ShapeDtypeStruct((Rp, D), jnp.float32),
        mesh=mesh,
        scratch_types=[
            pltpu.VMEM((rows_w,), jnp.int32),
            pltpu.VMEM((CH, D), jnp.float32),
            pltpu.SemaphoreType.DMA,
        ],
    )
    def k(x_hbm, tok_hbm, xs_hbm, idx_v, rows_v, sem):
        wid = lax.axis_index("s") * _NC + lax.axis_index("c")
        base = wid * rows_w
        pltpu.sync_copy(tok_hbm.at[pl.ds(base, rows_w)], idx_v)
        for c in range(nch):
            pltpu.async_copy(
                x_hbm.at[idx_v.at[pl.ds(c * CH, CH)]], rows_v, sem).wait()
            pltpu.sync_copy(rows_v, xs_hbm.at[pl.ds(base + c * CH, CH)])

    return k(x, sorted_tok)


def _ffn_body(be_ref, nb_ref, xs_ref, g_ref, w1_any, b1_any, w2_any, b2_any,
              out_ref, w1v, w2v, b1v, b2v, s1, s2, s3, s4):
    b = pl.program_id(0)
    e = be_ref[b]
    prev = jnp.where(b == 0, -1, be_ref[jnp.maximum(b - 1, 0)])

    @pl.when(e != prev)
    def _load():
        c1 = pltpu.make_async_copy(w1_any.at[e], w1v, s1)
        c2 = pltpu.make_async_copy(w2_any.at[e], w2v, s2)
        c3 = pltpu.make_async_copy(b1_any.at[pl.ds(e, 1)], b1v, s3)
        c4 = pltpu.make_async_copy(b2_any.at[pl.ds(e, 1)], b2v, s4)
        c1.start()
        c2.start()
        c3.start()
        c4.start()
        c1.wait()
        c2.wait()
        c3.wait()
        c4.wait()

    @pl.when(b < nb_ref[0])
    def _compute():
        xb = xs_ref[...]
        h = jnp.dot(xb, w1v[...], preferred_element_type=jnp.float32)
        h = jnp.maximum(h + b1v[...], 0.0)
        o = jnp.dot(h, w2v[...], preferred_element_type=jnp.float32)
        o = o + b2v[...]
        m = jnp.max(o, axis=1, keepdims=True)
        ex = jnp.exp(o - m)
        s = jnp.sum(ex, axis=1, keepdims=True)
        out_ref[...] = ex * (g_ref[...] / s)


def _ffn(xs, sorted_gate, W1, b1, W2, b2, block_expert, nb_active):
    Rp, D = xs.shape
    H = W1.shape[2]
    NB = Rp // BLK
    grid_spec = pltpu.PrefetchScalarGridSpec(
        num_scalar_prefetch=2,
        grid=(NB,),
        in_specs=[
            pl.BlockSpec((BLK, D), lambda b, be, nb: (b, 0)),
            pl.BlockSpec((BLK, 1), lambda b, be, nb: (b, 0)),
            pl.BlockSpec(memory_space=pltpu.ANY),
            pl.BlockSpec(memory_space=pltpu.ANY),
            pl.BlockSpec(memory_space=pltpu.ANY),
            pl.BlockSpec(memory_space=pltpu.ANY),
        ],
        out_specs=pl.BlockSpec((BLK, D), lambda b, be, nb: (b, 0)),
        scratch_shapes=[
            pltpu.VMEM((D, H), jnp.float32),
            pltpu.VMEM((H, D), jnp.float32),
            pltpu.VMEM((1, H), jnp.float32),
            pltpu.VMEM((1, D), jnp.float32),
            pltpu.SemaphoreType.DMA,
            pltpu.SemaphoreType.DMA,
            pltpu.SemaphoreType.DMA,
            pltpu.SemaphoreType.DMA,
        ],
    )
    return pl.pallas_call(
        _ffn_body,
        grid_spec=grid_spec,
        out_shape=jax.ShapeDtypeStruct((Rp, D), jnp.float32),
    )(block_expert, nb_active, xs, sorted_gate, W1, b1, W2, b2)


def _combine(wp, f1, f2, T):
    D = wp.shape[1]
    t_w = T // _NW
    CH = 32
    nch = t_w // CH
    mesh = plsc.VectorSubcoreMesh(core_axis_name="c", subcore_axis_name="s")

    @functools.partial(
        pl.kernel,
        out_type=jax.ShapeDtypeStruct((T, D), jnp.float32),
        mesh=mesh,
        scratch_types=[
            pltpu.VMEM((t_w,), jnp.int32),
            pltpu.VMEM((t_w,), jnp.int32),
            pltpu.VMEM((CH, D), jnp.float32),
            pltpu.VMEM((CH, D), jnp.float32),
            pltpu.SemaphoreType.DMA,
            pltpu.SemaphoreType.DMA,
        ],
    )
    def k(wp_hbm, f1_hbm, f2_hbm, y_hbm, i1v, i2v, buf1, buf2, sa, sb):
        wid = lax.axis_index("s") * _NC + lax.axis_index("c")
        base = wid * t_w
        pltpu.sync_copy(f1_hbm.at[pl.ds(base, t_w)], i1v)
        pltpu.sync_copy(f2_hbm.at[pl.ds(base, t_w)], i2v)
        nvec = D // 16
        for c in range(nch):
            cp1 = pltpu.async_copy(
                wp_hbm.at[i1v.at[pl.ds(c * CH, CH)]], buf1, sa)
            cp2 = pltpu.async_copy(
                wp_hbm.at[i2v.at[pl.ds(c * CH, CH)]], buf2, sb)
            cp1.wait()
            cp2.wait()

            def row_body(r, _):
                for j in range(nvec):
                    sl = pl.ds(16 * j, 16)
                    buf1[r, sl] = buf1[r, sl] + buf2[r, sl]
                return 0

            lax.fori_loop(0, CH, row_body, 0)
            pltpu.sync_copy(buf1, y_hbm.at[pl.ds(base + c * CH, CH)])

    return k(wp, f1, f2)


def kernel(x, w_gate, W1, b1, W2, b2):
    T, D = x.shape
    E = w_gate.shape[1]

    i1, i2, g1, g2, imp, load = _gating(x, w_gate)
    i1 = i1.reshape(T)
    i2 = i2.reshape(T)

    flat_e = jnp.concatenate([i1, i2])
    tok = jnp.tile(jnp.arange(T, dtype=jnp.int32), 2)
    onehot = (flat_e[:, None] == jnp.arange(E, dtype=jnp.int32)[None, :])
    csum = jnp.cumsum(onehot.astype(jnp.int32), axis=0)
    rank = jnp.take_along_axis(csum, flat_e[:, None], axis=1)[:, 0] - 1
    counts = csum[-1]
    padded = ((counts + BLK - 1) // BLK) * BLK
    ends = jnp.cumsum(padded)
    offs = ends - padded
    dest = (offs[flat_e] + rank).astype(jnp.int32)

    Rp = T * 2 + E * BLK
    NB = Rp // BLK
    sorted_tok = jnp.zeros((Rp,), jnp.int32).at[dest].set(tok)
    gflat = jnp.concatenate([g1.reshape(T), g2.reshape(T)])
    sorted_gate = jnp.zeros((Rp,), jnp.float32).at[dest].set(gflat)
    block_starts = jnp.arange(NB, dtype=jnp.int32) * BLK
    block_expert = jnp.sum(
        (block_starts[:, None] >= ends[None, :]).astype(jnp.int32), axis=1)
    block_expert = jnp.minimum(block_expert, E - 1).astype(jnp.int32)
    nb_active = (ends[-1] // BLK).astype(jnp.int32).reshape(1)

    xs = _dispatch(x, sorted_tok, Rp)
    wp = _ffn(xs, sorted_gate.reshape(Rp, 1), W1, b1, W2, b2,
              block_expert, nb_active)
    y = _combine(wp, dest[:T], dest[T:], T)

    eps = 1e-10
    imp = imp.reshape(E)
    load = load.reshape(E)
    cv_imp = jnp.var(imp, ddof=1) / (jnp.mean(imp) ** 2 + eps)
    cv_load = jnp.var(load, ddof=1) / (jnp.mean(load) ** 2 + eps)
    loss = (cv_imp + cv_load) * 0.01
    return (y, loss)


# speedup vs baseline: 1.8055x; 1.8055x over previous
import functools

import jax
import jax.numpy as jnp
from jax import lax
from jax.experimental import pallas as pl
from jax.experimental.pallas import tpu as pltpu
from jax.experimental.pallas import tpu_sc as plsc

BLK = 256
_NC, _NS = 2, 16
_NW = _NC * _NS


def _gating_body(x_ref, wg_ref, i1_ref, i2_ref, g1_ref, g2_ref, imp_ref,
                 load_ref):
    x = x_ref[...]
    wg = wg_ref[...]
    logits = jnp.dot(x, wg, preferred_element_type=jnp.float32)
    T, E = logits.shape
    iota_e = lax.broadcasted_iota(jnp.int32, (T, E), 1)
    m1 = jnp.max(logits, axis=1, keepdims=True)
    i1 = jnp.min(jnp.where(logits == m1, iota_e, E), axis=1, keepdims=True)
    masked = jnp.where(iota_e == i1, -jnp.inf, logits)
    m2 = jnp.max(masked, axis=1, keepdims=True)
    i2 = jnp.min(jnp.where(masked == m2, iota_e, E), axis=1, keepdims=True)
    e2 = jnp.exp(m2 - m1)
    den = 1.0 + e2
    g1 = 1.0 / den
    g2 = e2 / den
    i1_ref[...] = i1
    i2_ref[...] = i2
    g1_ref[...] = g1
    g2_ref[...] = g2
    oh1 = (iota_e == i1).astype(jnp.float32)
    oh2 = (iota_e == i2).astype(jnp.float32)
    imp_ref[...] = jnp.sum(oh1 * g1 + oh2 * g2, axis=0, keepdims=True)
    ld1 = jnp.where((iota_e == i1) & (g1 > 0), 1.0, 0.0)
    ld2 = jnp.where((iota_e == i2) & (g2 > 0), 1.0, 0.0)
    load_ref[...] = jnp.sum(ld1 + ld2, axis=0, keepdims=True)


def _gating(x, w_gate):
    T = x.shape[0]
    E = w_gate.shape[1]
    return pl.pallas_call(
        _gating_body,
        out_shape=[
            jax.ShapeDtypeStruct((T, 1), jnp.int32),
            jax.ShapeDtypeStruct((T, 1), jnp.int32),
            jax.ShapeDtypeStruct((T, 1), jnp.float32),
            jax.ShapeDtypeStruct((T, 1), jnp.float32),
            jax.ShapeDtypeStruct((1, E), jnp.float32),
            jax.ShapeDtypeStruct((1, E), jnp.float32),
        ],
    )(x, w_gate)


def _dispatch(x, sorted_tok, Rp):
    D = x.shape[1]
    rows_w = Rp // _NW
    CH = 64
    nch = rows_w // CH
    mesh = plsc.VectorSubcoreMesh(core_axis_name="c", subcore_axis_name="s", num_cores=_NC, num_subcores=_NS)

    @functools.partial(
        pl.kernel,
        out_type=jax.ShapeDtypeStruct((Rp, D), jnp.float32),
        mesh=mesh,
        scratch_types=[
            pltpu.VMEM((rows_w,), jnp.int32),
            pltpu.VMEM((CH, D), jnp.float32),
            pltpu.SemaphoreType.DMA,
        ],
    )
    def k(x_hbm, tok_hbm, xs_hbm, idx_v, rows_v, sem):
        wid = lax.axis_index("s") * _NC + lax.axis_index("c")
        base = wid * rows_w
        pltpu.sync_copy(tok_hbm.at[pl.ds(base, rows_w)], idx_v)
        for c in range(nch):
            pltpu.async_copy(
                x_hbm.at[idx_v.at[pl.ds(c * CH, CH)]], rows_v, sem).wait()
            pltpu.sync_copy(rows_v, xs_hbm.at[pl.ds(base + c * CH, CH)])

    return k(x, sorted_tok)


def _ffn_body(be_ref, nb_ref, xs_ref, g_ref, w1_any, b1_any, w2_any, b2_any,
              out_ref, w1v, w2v, b1v, b2v, s1, s2, s3, s4):
    b = pl.program_id(0)
    e = be_ref[b]
    prev = jnp.where(b == 0, -1, be_ref[jnp.maximum(b - 1, 0)])

    @pl.when(e != prev)
    def _load():
        c1 = pltpu.make_async_copy(w1_any.at[e], w1v, s1)
        c2 = pltpu.make_async_copy(w2_any.at[e], w2v, s2)
        c3 = pltpu.make_async_copy(b1_any.at[pl.ds(e, 1)], b1v, s3)
        c4 = pltpu.make_async_copy(b2_any.at[pl.ds(e, 1)], b2v, s4)
        c1.start()
        c2.start()
        c3.start()
        c4.start()
        c1.wait()
        c2.wait()
        c3.wait()
        c4.wait()

    @pl.when(b < nb_ref[0])
    def _compute():
        xb = xs_ref[...]
        h = jnp.dot(xb, w1v[...], preferred_element_type=jnp.float32)
        h = jnp.maximum(h + b1v[...], 0.0)
        o = jnp.dot(h, w2v[...], preferred_element_type=jnp.float32)
        o = o + b2v[...]
        m = jnp.max(o, axis=1, keepdims=True)
        ex = jnp.exp(o - m)
        s = jnp.sum(ex, axis=1, keepdims=True)
        out_ref[...] = ex * (g_ref[...] / s)


def _ffn(xs, sorted_gate, W1, b1, W2, b2, block_expert, nb_active):
    Rp, D = xs.shape
    H = W1.shape[2]
    NB = Rp // BLK
    grid_spec = pltpu.PrefetchScalarGridSpec(
        num_scalar_prefetch=2,
        grid=(NB,),
        in_specs=[
            pl.BlockSpec((BLK, D), lambda b, be, nb: (b, 0)),
            pl.BlockSpec((BLK, 1), lambda b, be, nb: (b, 0)),
            pl.BlockSpec(memory_space=pl.ANY),
            pl.BlockSpec(memory_space=pl.ANY),
            pl.BlockSpec(memory_space=pl.ANY),
            pl.BlockSpec(memory_space=pl.ANY),
        ],
        out_specs=pl.BlockSpec((BLK, D), lambda b, be, nb: (b, 0)),
        scratch_shapes=[
            pltpu.VMEM((D, H), jnp.float32),
            pltpu.VMEM((H, D), jnp.float32),
            pltpu.VMEM((1, H), jnp.float32),
            pltpu.VMEM((1, D), jnp.float32),
            pltpu.SemaphoreType.DMA,
            pltpu.SemaphoreType.DMA,
            pltpu.SemaphoreType.DMA,
            pltpu.SemaphoreType.DMA,
        ],
    )
    return pl.pallas_call(
        _ffn_body,
        grid_spec=grid_spec,
        out_shape=jax.ShapeDtypeStruct((Rp, D), jnp.float32),
    )(block_expert, nb_active, xs, sorted_gate, W1, b1, W2, b2)


def _combine(wp, f1, f2, T):
    D = wp.shape[1]
    t_w = T // _NW
    CH = 32
    nch = t_w // CH
    mesh = plsc.VectorSubcoreMesh(core_axis_name="c", subcore_axis_name="s", num_cores=_NC, num_subcores=_NS)

    @functools.partial(
        pl.kernel,
        out_type=jax.ShapeDtypeStruct((T, D), jnp.float32),
        mesh=mesh,
        scratch_types=[
            pltpu.VMEM((t_w,), jnp.int32),
            pltpu.VMEM((t_w,), jnp.int32),
            pltpu.VMEM((CH, D), jnp.float32),
            pltpu.VMEM((CH, D), jnp.float32),
            pltpu.SemaphoreType.DMA,
            pltpu.SemaphoreType.DMA,
        ],
    )
    def k(wp_hbm, f1_hbm, f2_hbm, y_hbm, i1v, i2v, buf1, buf2, sa, sb):
        wid = lax.axis_index("s") * _NC + lax.axis_index("c")
        base = wid * t_w
        pltpu.sync_copy(f1_hbm.at[pl.ds(base, t_w)], i1v)
        pltpu.sync_copy(f2_hbm.at[pl.ds(base, t_w)], i2v)
        nvec = D // 16
        for c in range(nch):
            cp1 = pltpu.async_copy(
                wp_hbm.at[i1v.at[pl.ds(c * CH, CH)]], buf1, sa)
            cp2 = pltpu.async_copy(
                wp_hbm.at[i2v.at[pl.ds(c * CH, CH)]], buf2, sb)
            cp1.wait()
            cp2.wait()

            def row_body(r, _):
                for j in range(nvec):
                    sl = pl.ds(16 * j, 16)
                    buf1[r, sl] = buf1[r, sl] + buf2[r, sl]
                return 0

            lax.fori_loop(0, CH, row_body, 0)
            pltpu.sync_copy(buf1, y_hbm.at[pl.ds(base + c * CH, CH)])

    return k(wp, f1, f2)


def kernel(x, w_gate, W1, b1, W2, b2):
    T, D = x.shape
    E = w_gate.shape[1]

    i1, i2, g1, g2, imp, load = _gating(x, w_gate)
    i1 = i1.reshape(T)
    i2 = i2.reshape(T)

    flat_e = jnp.concatenate([i1, i2])
    tok = jnp.tile(jnp.arange(T, dtype=jnp.int32), 2)
    onehot = (flat_e[:, None] == jnp.arange(E, dtype=jnp.int32)[None, :])
    csum = jnp.cumsum(onehot.astype(jnp.int32), axis=0)
    rank = jnp.take_along_axis(csum, flat_e[:, None], axis=1)[:, 0] - 1
    counts = csum[-1]
    padded = ((counts + BLK - 1) // BLK) * BLK
    ends = jnp.cumsum(padded)
    offs = ends - padded
    dest = (offs[flat_e] + rank).astype(jnp.int32)

    Rp = T * 2 + E * BLK
    NB = Rp // BLK
    sorted_tok = jnp.zeros((Rp,), jnp.int32).at[dest].set(tok)
    gflat = jnp.concatenate([g1.reshape(T), g2.reshape(T)])
    sorted_gate = jnp.zeros((Rp,), jnp.float32).at[dest].set(gflat)
    block_starts = jnp.arange(NB, dtype=jnp.int32) * BLK
    block_expert = jnp.sum(
        (block_starts[:, None] >= ends[None, :]).astype(jnp.int32), axis=1)
    block_expert = jnp.minimum(block_expert, E - 1).astype(jnp.int32)
    nb_active = (ends[-1] // BLK).astype(jnp.int32).reshape(1)

    xs = _dispatch(x, sorted_tok, Rp)
    wp = _ffn(xs, sorted_gate.reshape(Rp, 1), W1, b1, W2, b2,
              block_expert, nb_active)
    y = _combine(wp, dest[:T], dest[T:], T)

    eps = 1e-10
    imp = imp.reshape(E)
    load = load.reshape(E)
    cv_imp = jnp.var(imp, ddof=1) / (jnp.mean(imp) ** 2 + eps)
    cv_load = jnp.var(load, ddof=1) / (jnp.mean(load) ** 2 + eps)
    loss = (cv_imp + cv_load) * 0.01
    return (y, loss)


# speedup vs baseline: 2.7889x; 1.5447x over previous
import functools

import jax
import jax.numpy as jnp
from jax import lax
from jax.experimental import pallas as pl
from jax.experimental.pallas import tpu as pltpu
from jax.experimental.pallas import tpu_sc as plsc

BLK = 256
_NC, _NS = 2, 16
_NW = _NC * _NS


def _gating_body(x_ref, wg_ref, i1_ref, i2_ref, g1_ref, g2_ref, imp_ref,
                 load_ref):
    x = x_ref[...]
    wg = wg_ref[...]
    logits = jnp.dot(x, wg, preferred_element_type=jnp.float32)
    T, E = logits.shape
    iota_e = lax.broadcasted_iota(jnp.int32, (T, E), 1)
    m1 = jnp.max(logits, axis=1, keepdims=True)
    i1 = jnp.min(jnp.where(logits == m1, iota_e, E), axis=1, keepdims=True)
    masked = jnp.where(iota_e == i1, -jnp.inf, logits)
    m2 = jnp.max(masked, axis=1, keepdims=True)
    i2 = jnp.min(jnp.where(masked == m2, iota_e, E), axis=1, keepdims=True)
    e2 = jnp.exp(m2 - m1)
    den = 1.0 + e2
    g1 = 1.0 / den
    g2 = e2 / den
    i1_ref[...] = i1
    i2_ref[...] = i2
    g1_ref[...] = g1
    g2_ref[...] = g2
    oh1 = (iota_e == i1).astype(jnp.float32)
    oh2 = (iota_e == i2).astype(jnp.float32)
    imp_ref[...] = jnp.sum(oh1 * g1 + oh2 * g2, axis=0, keepdims=True)
    ld1 = jnp.where((iota_e == i1) & (g1 > 0), 1.0, 0.0)
    ld2 = jnp.where((iota_e == i2) & (g2 > 0), 1.0, 0.0)
    load_ref[...] = jnp.sum(ld1 + ld2, axis=0, keepdims=True)


def _gating(x, w_gate):
    T = x.shape[0]
    E = w_gate.shape[1]
    return pl.pallas_call(
        _gating_body,
        out_shape=[
            jax.ShapeDtypeStruct((T, 1), jnp.int32),
            jax.ShapeDtypeStruct((T, 1), jnp.int32),
            jax.ShapeDtypeStruct((T, 1), jnp.float32),
            jax.ShapeDtypeStruct((T, 1), jnp.float32),
            jax.ShapeDtypeStruct((1, E), jnp.float32),
            jax.ShapeDtypeStruct((1, E), jnp.float32),
        ],
    )(x, w_gate)


def _dispatch(x, dest3, Rp):
    T, D = x.shape
    NW, nch, CH = dest3.shape
    mesh = plsc.VectorSubcoreMesh(core_axis_name="c", subcore_axis_name="s",
                                  num_cores=_NC, num_subcores=_NS)

    @functools.partial(
        pl.kernel,
        out_type=jax.ShapeDtypeStruct((Rp, D), jnp.float32),
        mesh=mesh,
        scratch_types=[
            pltpu.VMEM((nch, CH), jnp.int32),
            pltpu.VMEM((CH, D), jnp.float32),
            pltpu.VMEM((CH, D), jnp.float32),
            pltpu.SemaphoreType.DMA,
            pltpu.SemaphoreType.DMA,
            pltpu.SemaphoreType.DMA,
            pltpu.SemaphoreType.DMA,
        ],
    )
    def k(x_hbm, dest_hbm, xs_hbm, idx_v, buf0, buf1, sr0, sr1, sw0, sw1):
        wid = lax.axis_index("s") * _NC + lax.axis_index("c")
        rows_w = nch * CH
        base = wid * rows_w
        tok0 = jnp.where(base >= T, base - T, base)
        pltpu.sync_copy(dest_hbm.at[wid], idx_v)
        bufs = (buf0, buf1)
        srs = (sr0, sr1)
        sws = (sw0, sw1)
        rd = pltpu.async_copy(x_hbm.at[pl.ds(tok0, CH)], bufs[0], srs[0])
        wrs = [None, None]
        for c in range(nch):
            i = c % 2
            rd.wait()
            wrs[i] = pltpu.async_copy(bufs[i], xs_hbm.at[idx_v.at[c]], sws[i])
            if c + 1 < nch:
                j = 1 - i
                if wrs[j] is not None:
                    wrs[j].wait()
                rd = pltpu.async_copy(
                    x_hbm.at[pl.ds(tok0 + (c + 1) * CH, CH)], bufs[j], srs[j])
        wrs[(nch - 1) % 2].wait()
        if nch > 1 and wrs[nch % 2] is not None:
            wrs[nch % 2].wait()

    return k(x, dest3)


def _ffn_body(be_ref, nb_ref, xs_ref, w1_ref, b1_ref, b2_ref, w2_any,
              out_ref, w2v, s2):
    b = pl.program_id(0)
    e = be_ref[b]
    prev = jnp.where(b == 0, -1, be_ref[jnp.maximum(b - 1, 0)])
    c2 = pltpu.make_async_copy(w2_any.at[e], w2v, s2)

    @pl.when(e != prev)
    def _load():
        c2.start()

    @pl.when(b < nb_ref[0])
    def _compute():
        xb = xs_ref[...]
        h = jnp.dot(xb, w1_ref[0], preferred_element_type=jnp.float32)
        h = jnp.maximum(h + b1_ref[0], 0.0)

        @pl.when(e != prev)
        def _wait():
            c2.wait()

        o = jnp.dot(h, w2v[...], preferred_element_type=jnp.float32)
        o = o + b2_ref[0]
        m = jnp.max(o, axis=1, keepdims=True)
        ex = jnp.exp(o - m)
        s = jnp.sum(ex, axis=1, keepdims=True)
        out_ref[...] = ex / s

    @pl.when((e != prev) & (b >= nb_ref[0]))
    def _wait_tail():
        c2.wait()


def _ffn(xs, W1, b1, W2, b2, block_expert, nb_active):
    Rp, D = xs.shape
    H = W1.shape[2]
    NB = Rp // BLK
    grid_spec = pltpu.PrefetchScalarGridSpec(
        num_scalar_prefetch=2,
        grid=(NB,),
        in_specs=[
            pl.BlockSpec((BLK, D), lambda b, be, nb: (b, 0)),
            pl.BlockSpec((1, D, H), lambda b, be, nb: (be[b], 0, 0)),
            pl.BlockSpec((1, 1, H), lambda b, be, nb: (be[b], 0, 0)),
            pl.BlockSpec((1, 1, D), lambda b, be, nb: (be[b], 0, 0)),
            pl.BlockSpec(memory_space=pl.ANY),
        ],
        out_specs=pl.BlockSpec((BLK, D), lambda b, be, nb: (b, 0)),
        scratch_shapes=[
            pltpu.VMEM((H, D), jnp.float32),
            pltpu.SemaphoreType.DMA,
        ],
    )
    return pl.pallas_call(
        _ffn_body,
        grid_spec=grid_spec,
        out_shape=jax.ShapeDtypeStruct((Rp, D), jnp.float32),
    )(block_expert, nb_active, xs, W1,
       b1.reshape(b1.shape[0], 1, b1.shape[1]),
       b2.reshape(b2.shape[0], 1, b2.shape[1]), W2)


def _combine(wp, f1, f2, g1, g2, T):
    D = wp.shape[1]
    t_w = T // _NW
    CH = 32
    nch = t_w // CH
    mesh = plsc.VectorSubcoreMesh(core_axis_name="c", subcore_axis_name="s",
                                  num_cores=_NC, num_subcores=_NS)

    @functools.partial(
        pl.kernel,
        out_type=jax.ShapeDtypeStruct((T, D), jnp.float32),
        mesh=mesh,
        scratch_types=[
            pltpu.VMEM((t_w,), jnp.int32),
            pltpu.VMEM((t_w,), jnp.int32),
            pltpu.VMEM((t_w, 16), jnp.float32),
            pltpu.VMEM((t_w, 16), jnp.float32),
            pltpu.VMEM((CH, D), jnp.float32),
            pltpu.VMEM((CH, D), jnp.float32),
            pltpu.SemaphoreType.DMA,
            pltpu.SemaphoreType.DMA,
        ],
    )
    def k(wp_hbm, f1_hbm, f2_hbm, g1_hbm, g2_hbm, y_hbm, i1v, i2v, gv1, gv2,
          buf1, buf2, sa, sb):
        wid = lax.axis_index("s") * _NC + lax.axis_index("c")
        base = wid * t_w
        pltpu.sync_copy(f1_hbm.at[pl.ds(base, t_w)], i1v)
        pltpu.sync_copy(f2_hbm.at[pl.ds(base, t_w)], i2v)
        pltpu.sync_copy(g1_hbm.at[pl.ds(base, t_w)], gv1)
        pltpu.sync_copy(g2_hbm.at[pl.ds(base, t_w)], gv2)
        nvec = D // 16
        for c in range(nch):
            cp1 = pltpu.async_copy(
                wp_hbm.at[i1v.at[pl.ds(c * CH, CH)]], buf1, sa)
            cp2 = pltpu.async_copy(
                wp_hbm.at[i2v.at[pl.ds(c * CH, CH)]], buf2, sb)
            cp1.wait()
            cp2.wait()

            def row_body(r, _):
                t_local = c * CH + r
                s1 = gv1[t_local, :]
                s2 = gv2[t_local, :]
                for j in range(nvec):
                    sl = pl.ds(16 * j, 16)
                    buf1[r, sl] = buf1[r, sl] * s1 + buf2[r, sl] * s2
                return 0

            lax.fori_loop(0, CH, row_body, 0)
            pltpu.sync_copy(buf1, y_hbm.at[pl.ds(base + c * CH, CH)])

    return k(wp, f1, f2, g1, g2)


def kernel(x, w_gate, W1, b1, W2, b2):
    T, D = x.shape
    E = w_gate.shape[1]

    i1, i2, g1, g2, imp, load = _gating(x, w_gate)
    i1 = i1.reshape(T)
    i2 = i2.reshape(T)

    flat_e = jnp.concatenate([i1, i2])
    onehot = (flat_e[:, None] == jnp.arange(E, dtype=jnp.int32)[None, :])
    oh32 = onehot.astype(jnp.int32)
    csum = jnp.cumsum(oh32, axis=0)
    rank = jnp.sum(csum * oh32, axis=1) - 1
    counts = csum[-1]
    padded = ((counts + BLK - 1) // BLK) * BLK
    ends = jnp.cumsum(padded)
    offs = ends - padded
    off_per_a = jnp.sum(offs[None, :] * oh32, axis=1)
    dest = (off_per_a + rank).astype(jnp.int32)

    Rp = T * 2 + E * BLK
    NB = Rp // BLK
    CH = 32
    nch = (2 * T) // (_NW * CH)
    dest3 = dest.reshape(_NW, nch, CH)
    block_starts = jnp.arange(NB, dtype=jnp.int32) * BLK
    block_expert = jnp.sum(
        (block_starts[:, None] >= ends[None, :]).astype(jnp.int32), axis=1)
    block_expert = jnp.minimum(block_expert, E - 1).astype(jnp.int32)
    nb_active = (ends[-1] // BLK).astype(jnp.int32).reshape(1)

    xs = _dispatch(x, dest3, Rp)
    wp = _ffn(xs, W1, b1, W2, b2, block_expert, nb_active)
    g1sp = jnp.broadcast_to(g1.reshape(T, 1), (T, 16))
    g2sp = jnp.broadcast_to(g2.reshape(T, 1), (T, 16))
    y = _combine(wp, dest[:T], dest[T:], g1sp, g2sp, T)

    eps = 1e-10
    imp = imp.reshape(E)
    load = load.reshape(E)
    cv_imp = jnp.var(imp, ddof=1) / (jnp.mean(imp) ** 2 + eps)
    cv_load = jnp.var(load, ddof=1) / (jnp.mean(load) ** 2 + eps)
    loss = (cv_imp + cv_load) * 0.01
    return (y, loss)


# speedup vs baseline: 2.9332x; 1.0517x over previous
import functools

import jax
import jax.numpy as jnp
from jax import lax
from jax.experimental import pallas as pl
from jax.experimental.pallas import tpu as pltpu
from jax.experimental.pallas import tpu_sc as plsc

BLK = 256
_NC, _NS = 2, 16
_NW = _NC * _NS


def _gating_body(x_ref, wg_ref, i1_ref, i2_ref, g1_ref, g2_ref, imp_ref,
                 load_ref):
    x = x_ref[...]
    wg = wg_ref[...]
    logits = jnp.dot(x, wg, preferred_element_type=jnp.float32)
    T, E = logits.shape
    iota_e = lax.broadcasted_iota(jnp.int32, (T, E), 1)
    m1 = jnp.max(logits, axis=1, keepdims=True)
    i1 = jnp.min(jnp.where(logits == m1, iota_e, E), axis=1, keepdims=True)
    masked = jnp.where(iota_e == i1, -jnp.inf, logits)
    m2 = jnp.max(masked, axis=1, keepdims=True)
    i2 = jnp.min(jnp.where(masked == m2, iota_e, E), axis=1, keepdims=True)
    e2 = jnp.exp(m2 - m1)
    den = 1.0 + e2
    g1 = 1.0 / den
    g2 = e2 / den
    i1_ref[...] = i1
    i2_ref[...] = i2
    g1_ref[...] = g1
    g2_ref[...] = g2
    oh1 = (iota_e == i1).astype(jnp.float32)
    oh2 = (iota_e == i2).astype(jnp.float32)
    imp_ref[...] = jnp.sum(oh1 * g1 + oh2 * g2, axis=0, keepdims=True)
    ld1 = jnp.where((iota_e == i1) & (g1 > 0), 1.0, 0.0)
    ld2 = jnp.where((iota_e == i2) & (g2 > 0), 1.0, 0.0)
    load_ref[...] = jnp.sum(ld1 + ld2, axis=0, keepdims=True)


def _gating(x, w_gate):
    T = x.shape[0]
    E = w_gate.shape[1]
    return pl.pallas_call(
        _gating_body,
        out_shape=[
            jax.ShapeDtypeStruct((T, 1), jnp.int32),
            jax.ShapeDtypeStruct((T, 1), jnp.int32),
            jax.ShapeDtypeStruct((T, 1), jnp.float32),
            jax.ShapeDtypeStruct((T, 1), jnp.float32),
            jax.ShapeDtypeStruct((1, E), jnp.float32),
            jax.ShapeDtypeStruct((1, E), jnp.float32),
        ],
    )(x, w_gate)


def _dispatch(x, dest3, Rp):
    T, D = x.shape
    NW, nch, CH = dest3.shape
    mesh = plsc.VectorSubcoreMesh(core_axis_name="c", subcore_axis_name="s",
                                  num_cores=_NC, num_subcores=_NS)

    @functools.partial(
        pl.kernel,
        out_type=jax.ShapeDtypeStruct((Rp, D), jnp.float32),
        mesh=mesh,
        scratch_types=[
            pltpu.VMEM((nch, CH), jnp.int32),
            pltpu.VMEM((CH, D), jnp.float32),
            pltpu.VMEM((CH, D), jnp.float32),
            pltpu.SemaphoreType.DMA,
            pltpu.SemaphoreType.DMA,
            pltpu.SemaphoreType.DMA,
            pltpu.SemaphoreType.DMA,
        ],
    )
    def k(x_hbm, dest_hbm, xs_hbm, idx_v, buf0, buf1, sr0, sr1, sw0, sw1):
        wid = lax.axis_index("s") * _NC + lax.axis_index("c")
        rows_w = nch * CH
        base = wid * rows_w
        tok0 = jnp.where(base >= T, base - T, base)
        pltpu.sync_copy(dest_hbm.at[wid], idx_v)
        bufs = (buf0, buf1)
        srs = (sr0, sr1)
        sws = (sw0, sw1)
        rd = pltpu.async_copy(x_hbm.at[pl.ds(tok0, CH)], bufs[0], srs[0])
        wrs = [None, None]
        for c in range(nch):
            i = c % 2
            rd.wait()
            wrs[i] = pltpu.async_copy(bufs[i], xs_hbm.at[idx_v.at[c]], sws[i])
            if c + 1 < nch:
                j = 1 - i
                if wrs[j] is not None:
                    wrs[j].wait()
                rd = pltpu.async_copy(
                    x_hbm.at[pl.ds(tok0 + (c + 1) * CH, CH)], bufs[j], srs[j])
        wrs[(nch - 1) % 2].wait()
        if nch > 1 and wrs[nch % 2] is not None:
            wrs[nch % 2].wait()

    return k(x, dest3)


def _ffn_body(be_ref, nb_ref, xs_ref, w1_ref, b1_ref, b2_ref, w2_any,
              out_ref, w2v, s2):
    b = pl.program_id(0)
    nb = nb_ref[0]
    last = pl.num_programs(0) - 1
    e = be_ref[b]
    prev = jnp.where(b == 0, -1, be_ref[jnp.maximum(b - 1, 0)])
    nxt = be_ref[jnp.minimum(b + 1, last)]
    c2_cur = pltpu.make_async_copy(w2_any.at[e], w2v, s2)
    c2_nxt = pltpu.make_async_copy(w2_any.at[nxt], w2v, s2)

    @pl.when(b == 0)
    def _load_first():
        c2_cur.start()

    @pl.when(b < nb)
    def _compute():
        xb = xs_ref[...]
        h = jnp.dot(xb, w1_ref[0], preferred_element_type=jnp.float32)
        h = jnp.maximum(h + b1_ref[0], 0.0)

        @pl.when(e != prev)
        def _wait():
            c2_cur.wait()

        o = jnp.dot(h, w2v[...], preferred_element_type=jnp.float32)

        @pl.when((nxt != e) & (b + 1 < nb))
        def _prefetch_next():
            c2_nxt.start()

        o = o + b2_ref[0]
        m = jnp.max(o, axis=1, keepdims=True)
        ex = jnp.exp(o - m)
        s = jnp.sum(ex, axis=1, keepdims=True)
        out_ref[...] = ex / s


def _ffn(xs, W1, b1, W2, b2, block_expert, nb_active):
    Rp, D = xs.shape
    H = W1.shape[2]
    NB = Rp // BLK
    grid_spec = pltpu.PrefetchScalarGridSpec(
        num_scalar_prefetch=2,
        grid=(NB,),
        in_specs=[
            pl.BlockSpec((BLK, D), lambda b, be, nb: (b, 0)),
            pl.BlockSpec((1, D, H), lambda b, be, nb: (be[b], 0, 0)),
            pl.BlockSpec((1, 1, H), lambda b, be, nb: (be[b], 0, 0)),
            pl.BlockSpec((1, 1, D), lambda b, be, nb: (be[b], 0, 0)),
            pl.BlockSpec(memory_space=pl.ANY),
        ],
        out_specs=pl.BlockSpec((BLK, D), lambda b, be, nb: (b, 0)),
        scratch_shapes=[
            pltpu.VMEM((H, D), jnp.float32),
            pltpu.SemaphoreType.DMA,
        ],
    )
    return pl.pallas_call(
        _ffn_body,
        grid_spec=grid_spec,
        out_shape=jax.ShapeDtypeStruct((Rp, D), jnp.float32),
    )(block_expert, nb_active, xs, W1,
       b1.reshape(b1.shape[0], 1, b1.shape[1]),
       b2.reshape(b2.shape[0], 1, b2.shape[1]), W2)


def _combine(wp, f1, f2, g1, g2, T):
    D = wp.shape[1]
    t_w = T // _NW
    CH = 16
    nch = t_w // CH
    mesh = plsc.VectorSubcoreMesh(core_axis_name="c", subcore_axis_name="s",
                                  num_cores=_NC, num_subcores=_NS)

    @functools.partial(
        pl.kernel,
        out_type=jax.ShapeDtypeStruct((T, D), jnp.float32),
        mesh=mesh,
        scratch_types=[
            pltpu.VMEM((t_w,), jnp.int32),
            pltpu.VMEM((t_w,), jnp.int32),
            pltpu.VMEM((t_w, 16), jnp.float32),
            pltpu.VMEM((t_w, 16), jnp.float32),
            pltpu.VMEM((CH, D), jnp.float32),
            pltpu.VMEM((CH, D), jnp.float32),
            pltpu.VMEM((CH, D), jnp.float32),
            pltpu.VMEM((CH, D), jnp.float32),
            pltpu.SemaphoreType.DMA,
            pltpu.SemaphoreType.DMA,
            pltpu.SemaphoreType.DMA,
            pltpu.SemaphoreType.DMA,
            pltpu.SemaphoreType.DMA,
            pltpu.SemaphoreType.DMA,
        ],
    )
    def k(wp_hbm, f1_hbm, f2_hbm, g1_hbm, g2_hbm, y_hbm, i1v, i2v, gv1, gv2,
          b1a, b2a, b1b, b2b, sa0, sb0, sa1, sb1, sw0, sw1):
        wid = lax.axis_index("s") * _NC + lax.axis_index("c")
        base = wid * t_w
        pltpu.sync_copy(f1_hbm.at[pl.ds(base, t_w)], i1v)
        pltpu.sync_copy(f2_hbm.at[pl.ds(base, t_w)], i2v)
        pltpu.sync_copy(g1_hbm.at[pl.ds(base, t_w)], gv1)
        pltpu.sync_copy(g2_hbm.at[pl.ds(base, t_w)], gv2)
        nvec = D // 16
        bufs1 = (b1a, b1b)
        bufs2 = (b2a, b2b)
        sas = (sa0, sa1)
        sbs = (sb0, sb1)
        sws = (sw0, sw1)

        def gathers(c, i):
            r1 = pltpu.async_copy(
                wp_hbm.at[i1v.at[pl.ds(c * CH, CH)]], bufs1[i], sas[i])
            r2 = pltpu.async_copy(
                wp_hbm.at[i2v.at[pl.ds(c * CH, CH)]], bufs2[i], sbs[i])
            return r1, r2

        rds = [None, None]
        wrs = [None, None]
        rds[0] = gathers(0, 0)
        for c in range(nch):
            i = c % 2
            rds[i][0].wait()
            rds[i][1].wait()
            if c + 1 < nch:
                j = 1 - i
                if wrs[j] is not None:
                    wrs[j].wait()
                rds[j] = gathers(c + 1, j)

            buf1 = bufs1[i]
            buf2 = bufs2[i]

            def row_body(r, _):
                t_local = c * CH + r
                s1 = gv1[t_local, :]
                s2 = gv2[t_local, :]
                for j2 in range(nvec):
                    sl = pl.ds(16 * j2, 16)
                    buf1[r, sl] = buf1[r, sl] * s1 + buf2[r, sl] * s2
                return 0

            lax.fori_loop(0, CH, row_body, 0)
            wrs[i] = pltpu.async_copy(
                buf1, y_hbm.at[pl.ds(base + c * CH, CH)], sws[i])
        wrs[(nch - 1) % 2].wait()
        if nch > 1 and wrs[nch % 2] is not None:
            wrs[nch % 2].wait()

    return k(wp, f1, f2, g1, g2)


def kernel(x, w_gate, W1, b1, W2, b2):
    T, D = x.shape
    E = w_gate.shape[1]

    i1, i2, g1, g2, imp, load = _gating(x, w_gate)
    i1 = i1.reshape(T)
    i2 = i2.reshape(T)

    flat_e = jnp.concatenate([i1, i2])
    onehot = (flat_e[:, None] == jnp.arange(E, dtype=jnp.int32)[None, :])
    oh32 = onehot.astype(jnp.int32)
    csum = jnp.cumsum(oh32, axis=0)
    rank = jnp.sum(csum * oh32, axis=1) - 1
    counts = csum[-1]
    padded = ((counts + BLK - 1) // BLK) * BLK
    ends = jnp.cumsum(padded)
    offs = ends - padded
    off_per_a = jnp.sum(offs[None, :] * oh32, axis=1)
    dest = (off_per_a + rank).astype(jnp.int32)

    Rp = T * 2 + E * BLK
    NB = Rp // BLK
    CH = 32
    nch = (2 * T) // (_NW * CH)
    dest3 = dest.reshape(_NW, nch, CH)
    block_starts = jnp.arange(NB, dtype=jnp.int32) * BLK
    block_expert = jnp.sum(
        (block_starts[:, None] >= ends[None, :]).astype(jnp.int32), axis=1)
    block_expert = jnp.minimum(block_expert, E - 1).astype(jnp.int32)
    nb_active = (ends[-1] // BLK).astype(jnp.int32).reshape(1)

    xs = _dispatch(x, dest3, Rp)
    wp = _ffn(xs, W1, b1, W2, b2, block_expert, nb_active)
    g1sp = jnp.broadcast_to(g1.reshape(T, 1), (T, 16))
    g2sp = jnp.broadcast_to(g2.reshape(T, 1), (T, 16))
    y = _combine(wp, dest[:T], dest[T:], g1sp, g2sp, T)

    eps = 1e-10
    imp = imp.reshape(E)
    load = load.reshape(E)
    cv_imp = jnp.var(imp, ddof=1) / (jnp.mean(imp) ** 2 + eps)
    cv_load = jnp.var(load, ddof=1) / (jnp.mean(load) ** 2 + eps)
    loss = (cv_imp + cv_load) * 0.01
    return (y, loss)


# speedup vs baseline: 2.9521x; 1.0065x over previous
import functools

import jax
import jax.numpy as jnp
from jax import lax
from jax.experimental import pallas as pl
from jax.experimental.pallas import tpu as pltpu
from jax.experimental.pallas import tpu_sc as plsc

BLK = 256
_NC, _NS = 2, 16
_NW = _NC * _NS


def _gating_body(x_ref, wg_ref, i1_ref, i2_ref, g1_ref, g2_ref, imp_ref,
                 load_ref):
    x = x_ref[...]
    wg = wg_ref[...]
    logits = jnp.dot(x, wg, preferred_element_type=jnp.float32)
    T, E = logits.shape
    iota_e = lax.broadcasted_iota(jnp.int32, (T, E), 1)
    m1 = jnp.max(logits, axis=1, keepdims=True)
    i1 = jnp.min(jnp.where(logits == m1, iota_e, E), axis=1, keepdims=True)
    masked = jnp.where(iota_e == i1, -jnp.inf, logits)
    m2 = jnp.max(masked, axis=1, keepdims=True)
    i2 = jnp.min(jnp.where(masked == m2, iota_e, E), axis=1, keepdims=True)
    e2 = jnp.exp(m2 - m1)
    den = 1.0 + e2
    g1 = 1.0 / den
    g2 = e2 / den
    i1_ref[...] = i1
    i2_ref[...] = i2
    g1_ref[...] = g1
    g2_ref[...] = g2
    oh1 = (iota_e == i1).astype(jnp.float32)
    oh2 = (iota_e == i2).astype(jnp.float32)
    imp_part = jnp.sum(oh1 * g1 + oh2 * g2, axis=0, keepdims=True)
    ld1 = jnp.where((iota_e == i1) & (g1 > 0), 1.0, 0.0)
    ld2 = jnp.where((iota_e == i2) & (g2 > 0), 1.0, 0.0)
    load_part = jnp.sum(ld1 + ld2, axis=0, keepdims=True)
    b = pl.program_id(0)

    @pl.when(b == 0)
    def _init():
        imp_ref[...] = imp_part
        load_ref[...] = load_part

    @pl.when(b > 0)
    def _acc():
        imp_ref[...] = imp_ref[...] + imp_part
        load_ref[...] = load_ref[...] + load_part


def _gating(x, w_gate):
    T = x.shape[0]
    D = x.shape[1]
    E = w_gate.shape[1]
    TB = 512
    nblk = T // TB
    return pl.pallas_call(
        _gating_body,
        grid=(nblk,),
        in_specs=[
            pl.BlockSpec((TB, D), lambda b: (b, 0)),
            pl.BlockSpec((D, E), lambda b: (0, 0)),
        ],
        out_specs=[
            pl.BlockSpec((TB, 1), lambda b: (b, 0)),
            pl.BlockSpec((TB, 1), lambda b: (b, 0)),
            pl.BlockSpec((TB, 1), lambda b: (b, 0)),
            pl.BlockSpec((TB, 1), lambda b: (b, 0)),
            pl.BlockSpec((1, E), lambda b: (0, 0)),
            pl.BlockSpec((1, E), lambda b: (0, 0)),
        ],
        out_shape=[
            jax.ShapeDtypeStruct((T, 1), jnp.int32),
            jax.ShapeDtypeStruct((T, 1), jnp.int32),
            jax.ShapeDtypeStruct((T, 1), jnp.float32),
            jax.ShapeDtypeStruct((T, 1), jnp.float32),
            jax.ShapeDtypeStruct((1, E), jnp.float32),
            jax.ShapeDtypeStruct((1, E), jnp.float32),
        ],
    )(x, w_gate)


def _dispatch(x, dest3, Rp):
    T, D = x.shape
    NW, nch, CH = dest3.shape
    mesh = plsc.VectorSubcoreMesh(core_axis_name="c", subcore_axis_name="s",
                                  num_cores=_NC, num_subcores=_NS)

    @functools.partial(
        pl.kernel,
        out_type=jax.ShapeDtypeStruct((Rp, D), jnp.float32),
        mesh=mesh,
        scratch_types=[
            pltpu.VMEM((nch, CH), jnp.int32),
            pltpu.VMEM((CH, D), jnp.float32),
            pltpu.VMEM((CH, D), jnp.float32),
            pltpu.SemaphoreType.DMA,
            pltpu.SemaphoreType.DMA,
            pltpu.SemaphoreType.DMA,
            pltpu.SemaphoreType.DMA,
        ],
    )
    def k(x_hbm, dest_hbm, xs_hbm, idx_v, buf0, buf1, sr0, sr1, sw0, sw1):
        wid = lax.axis_index("s") * _NC + lax.axis_index("c")
        rows_w = nch * CH
        base = wid * rows_w
        tok0 = jnp.where(base >= T, base - T, base)
        pltpu.sync_copy(dest_hbm.at[wid], idx_v)
        bufs = (buf0, buf1)
        srs = (sr0, sr1)
        sws = (sw0, sw1)
        rd = pltpu.async_copy(x_hbm.at[pl.ds(tok0, CH)], bufs[0], srs[0])
        wrs = [None, None]
        for c in range(nch):
            i = c % 2
            rd.wait()
            wrs[i] = pltpu.async_copy(bufs[i], xs_hbm.at[idx_v.at[c]], sws[i])
            if c + 1 < nch:
                j = 1 - i
                if wrs[j] is not None:
                    wrs[j].wait()
                rd = pltpu.async_copy(
                    x_hbm.at[pl.ds(tok0 + (c + 1) * CH, CH)], bufs[j], srs[j])
        wrs[(nch - 1) % 2].wait()
        if nch > 1 and wrs[nch % 2] is not None:
            wrs[nch % 2].wait()

    return k(x, dest3)


_NQ = 4


def _w1_copies(w1_any, e, w1v, sems, D):
    ck = D // _NQ
    return [pltpu.make_async_copy(
        w1_any.at[e, pl.ds(q * ck, ck), :], w1v.at[pl.ds(q * ck, ck), :],
        sems.at[q]) for q in range(_NQ)]


def _w2_copies(w2_any, e, w2v, sems, H):
    ck = H // _NQ
    return [pltpu.make_async_copy(
        w2_any.at[e, pl.ds(q * ck, ck), :], w2v.at[pl.ds(q * ck, ck), :],
        sems.at[q]) for q in range(_NQ)]


def _ffn_body(be_ref, nb_ref, slot_ref, start_ref, nre_ref,
              xs_ref, b1_ref, b2_ref, w1_any, w2_any,
              out_ref, w1a, w1b, w2v, s1a, s1b, s2):
    D, H = w1a.shape
    b = pl.program_id(0)
    nb = nb_ref[0]
    last = pl.num_programs(0) - 1
    e = be_ref[b]
    prev = jnp.where(b == 0, -1, be_ref[jnp.maximum(b - 1, 0)])
    nxt = be_ref[jnp.minimum(b + 1, last)]
    slot = slot_ref[b]
    nre = nre_ref[b]

    @pl.when(b == 0)
    def _load_first():
        for c in _w1_copies(w1_any, e, w1a, s1a, D):
            c.start()
        for c in _w2_copies(w2_any, e, w2v, s2, H):
            c.start()

    is_first = (e != prev) & (b < nb)

    @pl.when(is_first & (slot == 0))
    def _wait_w1a():
        for c in _w1_copies(w1_any, e, w1a, s1a, D):
            c.wait()

    @pl.when(is_first & (slot == 1))
    def _wait_w1b():
        for c in _w1_copies(w1_any, e, w1b, s1b, D):
            c.wait()

    def _compute(w1_cur, w1_nxt, s1_nxt):
        xb = xs_ref[...]
        h = jnp.dot(xb, w1_cur[...], preferred_element_type=jnp.float32)
        h = jnp.maximum(h + b1_ref[0], 0.0)

        @pl.when(e != prev)
        def _wait_w2():
            for c in _w2_copies(w2_any, e, w2v, s2, H):
                c.wait()

        @pl.when(start_ref[b] == 1)
        def _start_w1_next():
            for c in _w1_copies(w1_any, nre, w1_nxt, s1_nxt, D):
                c.start()

        o = jnp.dot(h, w2v[...], preferred_element_type=jnp.float32)

        @pl.when((nxt != e) & (b + 1 < nb))
        def _start_w2_next():
            for c in _w2_copies(w2_any, nxt, w2v, s2, H):
                c.start()

        o = o + b2_ref[0]
        m = jnp.max(o, axis=1, keepdims=True)
        ex = jnp.exp(o - m)
        s = jnp.sum(ex, axis=1, keepdims=True)
        out_ref[...] = ex / s

    @pl.when((b < nb) & (slot == 0))
    def _compute0():
        _compute(w1a, w1b, s1b)

    @pl.when((b < nb) & (slot == 1))
    def _compute1():
        _compute(w1b, w1a, s1a)


def _ffn(xs, W1, b1, W2, b2, block_expert, nb_active, slot_arr, start_arr,
         nre_arr):
    Rp, D = xs.shape
    H = W1.shape[2]
    NB = Rp // BLK
    grid_spec = pltpu.PrefetchScalarGridSpec(
        num_scalar_prefetch=5,
        grid=(NB,),
        in_specs=[
            pl.BlockSpec((BLK, D), lambda b, *_: (b, 0)),
            pl.BlockSpec((1, 1, H), lambda b, be, nb, sl, st, nr: (be[b], 0, 0)),
            pl.BlockSpec((1, 1, D), lambda b, be, nb, sl, st, nr: (be[b], 0, 0)),
            pl.BlockSpec(memory_space=pl.ANY),
            pl.BlockSpec(memory_space=pl.ANY),
        ],
        out_specs=pl.BlockSpec((BLK, D), lambda b, *_: (b, 0)),
        scratch_shapes=[
            pltpu.VMEM((D, H), jnp.float32),
            pltpu.VMEM((D, H), jnp.float32),
            pltpu.VMEM((H, D), jnp.float32),
            pltpu.SemaphoreType.DMA((_NQ,)),
            pltpu.SemaphoreType.DMA((_NQ,)),
            pltpu.SemaphoreType.DMA((_NQ,)),
        ],
    )
    return pl.pallas_call(
        _ffn_body,
        grid_spec=grid_spec,
        out_shape=jax.ShapeDtypeStruct((Rp, D), jnp.float32),
    )(block_expert, nb_active, slot_arr, start_arr, nre_arr, xs,
       b1.reshape(b1.shape[0], 1, b1.shape[1]),
       b2.reshape(b2.shape[0], 1, b2.shape[1]), W1, W2)


def _combine(wp, f1, f2, g1, g2, T):
    D = wp.shape[1]
    t_w = T // _NW
    CH = 16
    nch = t_w // CH
    mesh = plsc.VectorSubcoreMesh(core_axis_name="c", subcore_axis_name="s",
                                  num_cores=_NC, num_subcores=_NS)

    @functools.partial(
        pl.kernel,
        out_type=jax.ShapeDtypeStruct((T, D), jnp.float32),
        mesh=mesh,
        scratch_types=[
            pltpu.VMEM((t_w,), jnp.int32),
            pltpu.VMEM((t_w,), jnp.int32),
            pltpu.VMEM((t_w, 16), jnp.float32),
            pltpu.VMEM((t_w, 16), jnp.float32),
            pltpu.VMEM((CH, D), jnp.float32),
            pltpu.VMEM((CH, D), jnp.float32),
            pltpu.VMEM((CH, D), jnp.float32),
            pltpu.VMEM((CH, D), jnp.float32),
            pltpu.SemaphoreType.DMA,
            pltpu.SemaphoreType.DMA,
            pltpu.SemaphoreType.DMA,
            pltpu.SemaphoreType.DMA,
            pltpu.SemaphoreType.DMA,
            pltpu.SemaphoreType.DMA,
        ],
    )
    def k(wp_hbm, f1_hbm, f2_hbm, g1_hbm, g2_hbm, y_hbm, i1v, i2v, gv1, gv2,
          b1a, b2a, b1b, b2b, sa0, sb0, sa1, sb1, sw0, sw1):
        wid = lax.axis_index("s") * _NC + lax.axis_index("c")
        base = wid * t_w
        pltpu.sync_copy(f1_hbm.at[pl.ds(base, t_w)], i1v)
        pltpu.sync_copy(f2_hbm.at[pl.ds(base, t_w)], i2v)
        pltpu.sync_copy(g1_hbm.at[pl.ds(base, t_w)], gv1)
        pltpu.sync_copy(g2_hbm.at[pl.ds(base, t_w)], gv2)
        nvec = D // 16
        bufs1 = (b1a, b1b)
        bufs2 = (b2a, b2b)
        sas = (sa0, sa1)
        sbs = (sb0, sb1)
        sws = (sw0, sw1)

        def gathers(c, i):
            r1 = pltpu.async_copy(
                wp_hbm.at[i1v.at[pl.ds(c * CH, CH)]], bufs1[i], sas[i])
            r2 = pltpu.async_copy(
                wp_hbm.at[i2v.at[pl.ds(c * CH, CH)]], bufs2[i], sbs[i])
            return r1, r2

        rds = [None, None]
        wrs = [None, None]
        rds[0] = gathers(0, 0)
        for c in range(nch):
            i = c % 2
            rds[i][0].wait()
            rds[i][1].wait()
            if c + 1 < nch:
                j = 1 - i
                if wrs[j] is not None:
                    wrs[j].wait()
                rds[j] = gathers(c + 1, j)

            buf1 = bufs1[i]
            buf2 = bufs2[i]

            def row_body(r, _):
                t_local = c * CH + r
                s1 = gv1[t_local, :]
                s2 = gv2[t_local, :]
                for j2 in range(nvec):
                    sl = pl.ds(16 * j2, 16)
                    buf1[r, sl] = buf1[r, sl] * s1 + buf2[r, sl] * s2
                return 0

            lax.fori_loop(0, CH, row_body, 0)
            wrs[i] = pltpu.async_copy(
                buf1, y_hbm.at[pl.ds(base + c * CH, CH)], sws[i])
        wrs[(nch - 1) % 2].wait()
        if nch > 1 and wrs[nch % 2] is not None:
            wrs[nch % 2].wait()

    return k(wp, f1, f2, g1, g2)


def kernel(x, w_gate, W1, b1, W2, b2):
    T, D = x.shape
    E = w_gate.shape[1]

    i1, i2, g1, g2, imp, load = _gating(x, w_gate)
    i1 = i1.reshape(T)
    i2 = i2.reshape(T)

    flat_e = jnp.concatenate([i1, i2])
    onehot = (flat_e[:, None] == jnp.arange(E, dtype=jnp.int32)[None, :])
    oh32 = onehot.astype(jnp.int32)
    csum = jnp.cumsum(oh32, axis=0)
    rank = jnp.sum(csum * oh32, axis=1) - 1
    counts = csum[-1]
    padded = ((counts + BLK - 1) // BLK) * BLK
    ends = jnp.cumsum(padded)
    offs = ends - padded
    off_per_a = jnp.sum(offs[None, :] * oh32, axis=1)
    dest = (off_per_a + rank).astype(jnp.int32)

    Rp = T * 2 + E * BLK
    NB = Rp // BLK
    CH = 32
    nch = (2 * T) // (_NW * CH)
    dest3 = dest.reshape(_NW, nch, CH)
    block_starts = jnp.arange(NB, dtype=jnp.int32) * BLK
    block_expert = jnp.sum(
        (block_starts[:, None] >= ends[None, :]).astype(jnp.int32), axis=1)
    block_expert = jnp.minimum(block_expert, E - 1).astype(jnp.int32)
    nb_active = (ends[-1] // BLK).astype(jnp.int32).reshape(1)

    be = block_expert
    is_first = jnp.concatenate(
        [jnp.ones((1,), jnp.bool_), be[1:] != be[:-1]])
    run_id = jnp.cumsum(is_first.astype(jnp.int32)) - 1
    slot_arr = (run_id % 2).astype(jnp.int32)
    iota_nb = jnp.arange(NB, dtype=jnp.int32)
    arr = jnp.where(is_first, iota_nb, NB)
    min_from = lax.cummin(arr[::-1])[::-1]
    nxt_t = jnp.concatenate([min_from[1:], jnp.full((1,), NB, jnp.int32)])
    nre_arr = be[jnp.minimum(nxt_t, NB - 1)].astype(jnp.int32)
    start_arr = (is_first & (nxt_t < nb_active[0])).astype(jnp.int32)

    xs = _dispatch(x, dest3, Rp)
    wp = _ffn(xs, W1, b1, W2, b2, block_expert, nb_active, slot_arr,
              start_arr, nre_arr)
    g1sp = jnp.broadcast_to(g1.reshape(T, 1), (T, 16))
    g2sp = jnp.broadcast_to(g2.reshape(T, 1), (T, 16))
    y = _combine(wp, dest[:T], dest[T:], g1sp, g2sp, T)

    eps = 1e-10
    imp = imp.reshape(E)
    load = load.reshape(E)
    cv_imp = jnp.var(imp, ddof=1) / (jnp.mean(imp) ** 2 + eps)
    cv_load = jnp.var(load, ddof=1) / (jnp.mean(load) ** 2 + eps)
    loss = (cv_imp + cv_load) * 0.01
    return (y, loss)


# speedup vs baseline: 2.9873x; 1.0119x over previous
import functools

import jax
import jax.numpy as jnp
from jax import lax
from jax.experimental import pallas as pl
from jax.experimental.pallas import tpu as pltpu
from jax.experimental.pallas import tpu_sc as plsc

BLK = 256
_NC, _NS = 2, 16
_NW = _NC * _NS


def _gating_body(x_ref, wg_ref, i1_ref, i2_ref, g1_ref, g2_ref, imp_ref,
                 load_ref):
    x = x_ref[...]
    wg = wg_ref[...]
    logits = jnp.dot(x, wg, preferred_element_type=jnp.float32)
    T, E = logits.shape
    iota_e = lax.broadcasted_iota(jnp.int32, (T, E), 1)
    m1 = jnp.max(logits, axis=1, keepdims=True)
    i1 = jnp.min(jnp.where(logits == m1, iota_e, E), axis=1, keepdims=True)
    masked = jnp.where(iota_e == i1, -jnp.inf, logits)
    m2 = jnp.max(masked, axis=1, keepdims=True)
    i2 = jnp.min(jnp.where(masked == m2, iota_e, E), axis=1, keepdims=True)
    e2 = jnp.exp(m2 - m1)
    den = 1.0 + e2
    g1 = 1.0 / den
    g2 = e2 / den
    i1_ref[...] = i1
    i2_ref[...] = i2
    g1_ref[...] = jnp.broadcast_to(g1, (T, 16))
    g2_ref[...] = jnp.broadcast_to(g2, (T, 16))
    oh1 = (iota_e == i1).astype(jnp.float32)
    oh2 = (iota_e == i2).astype(jnp.float32)
    imp_part = jnp.sum(oh1 * g1 + oh2 * g2, axis=0, keepdims=True)
    ld1 = jnp.where((iota_e == i1) & (g1 > 0), 1.0, 0.0)
    ld2 = jnp.where((iota_e == i2) & (g2 > 0), 1.0, 0.0)
    load_part = jnp.sum(ld1 + ld2, axis=0, keepdims=True)
    b = pl.program_id(0)

    @pl.when(b == 0)
    def _init():
        imp_ref[...] = imp_part
        load_ref[...] = load_part

    @pl.when(b > 0)
    def _acc():
        imp_ref[...] = imp_ref[...] + imp_part
        load_ref[...] = load_ref[...] + load_part


def _gating(x, w_gate):
    T = x.shape[0]
    D = x.shape[1]
    E = w_gate.shape[1]
    TB = 512
    nblk = T // TB
    return pl.pallas_call(
        _gating_body,
        grid=(nblk,),
        in_specs=[
            pl.BlockSpec((TB, D), lambda b: (b, 0)),
            pl.BlockSpec((D, E), lambda b: (0, 0)),
        ],
        out_specs=[
            pl.BlockSpec((TB, 1), lambda b: (b, 0)),
            pl.BlockSpec((TB, 1), lambda b: (b, 0)),
            pl.BlockSpec((TB, 16), lambda b: (b, 0)),
            pl.BlockSpec((TB, 16), lambda b: (b, 0)),
            pl.BlockSpec((1, E), lambda b: (0, 0)),
            pl.BlockSpec((1, E), lambda b: (0, 0)),
        ],
        out_shape=[
            jax.ShapeDtypeStruct((T, 1), jnp.int32),
            jax.ShapeDtypeStruct((T, 1), jnp.int32),
            jax.ShapeDtypeStruct((T, 16), jnp.float32),
            jax.ShapeDtypeStruct((T, 16), jnp.float32),
            jax.ShapeDtypeStruct((1, E), jnp.float32),
            jax.ShapeDtypeStruct((1, E), jnp.float32),
        ],
    )(x, w_gate)


def _dispatch(x, dest3, Rp):
    T, D = x.shape
    NW, nch, CH = dest3.shape
    mesh = plsc.VectorSubcoreMesh(core_axis_name="c", subcore_axis_name="s",
                                  num_cores=_NC, num_subcores=_NS)

    @functools.partial(
        pl.kernel,
        out_type=jax.ShapeDtypeStruct((Rp, D), jnp.float32),
        mesh=mesh,
        scratch_types=[
            pltpu.VMEM((nch, CH), jnp.int32),
            pltpu.VMEM((CH, D), jnp.float32),
            pltpu.VMEM((CH, D), jnp.float32),
            pltpu.VMEM((CH, D), jnp.float32),
            pltpu.SemaphoreType.DMA,
            pltpu.SemaphoreType.DMA,
            pltpu.SemaphoreType.DMA,
            pltpu.SemaphoreType.DMA,
            pltpu.SemaphoreType.DMA,
            pltpu.SemaphoreType.DMA,
        ],
    )
    def k(x_hbm, dest_hbm, xs_hbm, idx_v, buf0, buf1, buf2,
          sr0, sr1, sr2, sw0, sw1, sw2):
        wid = lax.axis_index("s") * _NC + lax.axis_index("c")
        rows_w = nch * CH
        base = wid * rows_w
        tok0 = jnp.where(base >= T, base - T, base)
        pltpu.sync_copy(dest_hbm.at[wid], idx_v)
        nbuf = 3
        bufs = (buf0, buf1, buf2)
        srs = (sr0, sr1, sr2)
        sws = (sw0, sw1, sw2)
        rd = pltpu.async_copy(x_hbm.at[pl.ds(tok0, CH)], bufs[0], srs[0])
        wrs = [None] * nbuf
        for c in range(nch):
            i = c % nbuf
            rd.wait()
            wrs[i] = pltpu.async_copy(bufs[i], xs_hbm.at[idx_v.at[c]], sws[i])
            if c + 1 < nch:
                j = (c + 1) % nbuf
                if wrs[j] is not None:
                    wrs[j].wait()
                rd = pltpu.async_copy(
                    x_hbm.at[pl.ds(tok0 + (c + 1) * CH, CH)], bufs[j], srs[j])
        for w in wrs:
            if w is not None:
                w.wait()

    return k(x, dest3)


_NQ = 4


def _w1_copies(w1_any, e, w1v, sems, D):
    ck = D // _NQ
    return [pltpu.make_async_copy(
        w1_any.at[e, pl.ds(q * ck, ck), :], w1v.at[pl.ds(q * ck, ck), :],
        sems.at[q]) for q in range(_NQ)]


def _w2_copies(w2_any, e, w2v, sems, H):
    ck = H // _NQ
    return [pltpu.make_async_copy(
        w2_any.at[e, pl.ds(q * ck, ck), :], w2v.at[pl.ds(q * ck, ck), :],
        sems.at[q]) for q in range(_NQ)]


def _ffn_body(be_ref, nb_ref, slot_ref, start_ref, nre_ref,
              xs_ref, b1_ref, b2_ref, w1_any, w2_any,
              out_ref, w1a, w1b, w2v, s1a, s1b, s2):
    D, H = w1a.shape
    b = pl.program_id(0)
    nb = nb_ref[0]
    last = pl.num_programs(0) - 1
    e = be_ref[b]
    prev = jnp.where(b == 0, -1, be_ref[jnp.maximum(b - 1, 0)])
    nxt = be_ref[jnp.minimum(b + 1, last)]
    slot = slot_ref[b]
    nre = nre_ref[b]

    @pl.when(b == 0)
    def _load_first():
        for c in _w1_copies(w1_any, e, w1a, s1a, D):
            c.start()
        for c in _w2_copies(w2_any, e, w2v, s2, H):
            c.start()

    is_first = (e != prev) & (b < nb)

    @pl.when(is_first & (slot == 0))
    def _wait_w1a():
        for c in _w1_copies(w1_any, e, w1a, s1a, D):
            c.wait()

    @pl.when(is_first & (slot == 1))
    def _wait_w1b():
        for c in _w1_copies(w1_any, e, w1b, s1b, D):
            c.wait()

    def _compute(w1_cur, w1_nxt, s1_nxt):
        xb = xs_ref[...]
        h = jnp.dot(xb, w1_cur[...], preferred_element_type=jnp.float32)
        h = jnp.maximum(h + b1_ref[0], 0.0)

        @pl.when(e != prev)
        def _wait_w2():
            for c in _w2_copies(w2_any, e, w2v, s2, H):
                c.wait()

        @pl.when(start_ref[b] == 1)
        def _start_w1_next():
            for c in _w1_copies(w1_any, nre, w1_nxt, s1_nxt, D):
                c.start()

        o = jnp.dot(h, w2v[...], preferred_element_type=jnp.float32)

        @pl.when((nxt != e) & (b + 1 < nb))
        def _start_w2_next():
            for c in _w2_copies(w2_any, nxt, w2v, s2, H):
                c.start()

        o = o + b2_ref[0]
        m = jnp.max(o, axis=1, keepdims=True)
        ex = jnp.exp(o - m)
        s = jnp.sum(ex, axis=1, keepdims=True)
        out_ref[...] = ex / s

    @pl.when((b < nb) & (slot == 0))
    def _compute0():
        _compute(w1a, w1b, s1b)

    @pl.when((b < nb) & (slot == 1))
    def _compute1():
        _compute(w1b, w1a, s1a)


def _ffn(xs, W1, b1, W2, b2, block_expert, nb_active, slot_arr, start_arr,
         nre_arr):
    Rp, D = xs.shape
    H = W1.shape[2]
    NB = Rp // BLK
    grid_spec = pltpu.PrefetchScalarGridSpec(
        num_scalar_prefetch=5,
        grid=(NB,),
        in_specs=[
            pl.BlockSpec((BLK, D), lambda b, *_: (b, 0)),
            pl.BlockSpec((1, 1, H), lambda b, be, nb, sl, st, nr: (be[b], 0, 0)),
            pl.BlockSpec((1, 1, D), lambda b, be, nb, sl, st, nr: (be[b], 0, 0)),
            pl.BlockSpec(memory_space=pl.ANY),
            pl.BlockSpec(memory_space=pl.ANY),
        ],
        out_specs=pl.BlockSpec((BLK, D), lambda b, *_: (b, 0)),
        scratch_shapes=[
            pltpu.VMEM((D, H), jnp.float32),
            pltpu.VMEM((D, H), jnp.float32),
            pltpu.VMEM((H, D), jnp.float32),
            pltpu.SemaphoreType.DMA((_NQ,)),
            pltpu.SemaphoreType.DMA((_NQ,)),
            pltpu.SemaphoreType.DMA((_NQ,)),
        ],
    )
    return pl.pallas_call(
        _ffn_body,
        grid_spec=grid_spec,
        out_shape=jax.ShapeDtypeStruct((Rp, D), jnp.float32),
    )(block_expert, nb_active, slot_arr, start_arr, nre_arr, xs,
       b1.reshape(b1.shape[0], 1, b1.shape[1]),
       b2.reshape(b2.shape[0], 1, b2.shape[1]), W1, W2)


def _combine(wp, f1, f2, g1, g2, T):
    D = wp.shape[1]
    t_w = T // _NW
    CH = 16
    nch = t_w // CH
    mesh = plsc.VectorSubcoreMesh(core_axis_name="c", subcore_axis_name="s",
                                  num_cores=_NC, num_subcores=_NS)

    @functools.partial(
        pl.kernel,
        out_type=jax.ShapeDtypeStruct((T, D), jnp.float32),
        mesh=mesh,
        scratch_types=[
            pltpu.VMEM((t_w,), jnp.int32),
            pltpu.VMEM((t_w,), jnp.int32),
            pltpu.VMEM((t_w, 16), jnp.float32),
            pltpu.VMEM((t_w, 16), jnp.float32),
            pltpu.VMEM((CH, D), jnp.float32),
            pltpu.VMEM((CH, D), jnp.float32),
            pltpu.VMEM((CH, D), jnp.float32),
            pltpu.VMEM((CH, D), jnp.float32),
            pltpu.SemaphoreType.DMA,
            pltpu.SemaphoreType.DMA,
            pltpu.SemaphoreType.DMA,
            pltpu.SemaphoreType.DMA,
            pltpu.SemaphoreType.DMA,
            pltpu.SemaphoreType.DMA,
        ],
    )
    def k(wp_hbm, f1_hbm, f2_hbm, g1_hbm, g2_hbm, y_hbm, i1v, i2v, gv1, gv2,
          b1a, b2a, b1b, b2b, sa0, sb0, sa1, sb1, sw0, sw1):
        wid = lax.axis_index("s") * _NC + lax.axis_index("c")
        base = wid * t_w
        pltpu.sync_copy(f1_hbm.at[pl.ds(base, t_w)], i1v)
        pltpu.sync_copy(f2_hbm.at[pl.ds(base, t_w)], i2v)
        pltpu.sync_copy(g1_hbm.at[pl.ds(base, t_w)], gv1)
        pltpu.sync_copy(g2_hbm.at[pl.ds(base, t_w)], gv2)
        nvec = D // 16
        bufs1 = (b1a, b1b)
        bufs2 = (b2a, b2b)
        sas = (sa0, sa1)
        sbs = (sb0, sb1)
        sws = (sw0, sw1)

        def gathers(c, i):
            r1 = pltpu.async_copy(
                wp_hbm.at[i1v.at[pl.ds(c * CH, CH)]], bufs1[i], sas[i])
            r2 = pltpu.async_copy(
                wp_hbm.at[i2v.at[pl.ds(c * CH, CH)]], bufs2[i], sbs[i])
            return r1, r2

        rds = [None, None]
        wrs = [None, None]
        rds[0] = gathers(0, 0)
        for c in range(nch):
            i = c % 2
            rds[i][0].wait()
            rds[i][1].wait()
            if c + 1 < nch:
                j = 1 - i
                if wrs[j] is not None:
                    wrs[j].wait()
                rds[j] = gathers(c + 1, j)

            buf1 = bufs1[i]
            buf2 = bufs2[i]

            def row_body(r, _):
                t_local = c * CH + r
                s1 = gv1[t_local, :]
                s2 = gv2[t_local, :]
                for j2 in range(nvec):
                    sl = pl.ds(16 * j2, 16)
                    buf1[r, sl] = buf1[r, sl] * s1 + buf2[r, sl] * s2
                return 0

            lax.fori_loop(0, CH, row_body, 0)
            wrs[i] = pltpu.async_copy(
                buf1, y_hbm.at[pl.ds(base + c * CH, CH)], sws[i])
        wrs[(nch - 1) % 2].wait()
        if nch > 1 and wrs[nch % 2] is not None:
            wrs[nch % 2].wait()

    return k(wp, f1, f2, g1, g2)


def kernel(x, w_gate, W1, b1, W2, b2):
    T, D = x.shape
    E = w_gate.shape[1]

    i1, i2, g1sp, g2sp, imp, load = _gating(x, w_gate)
    i1 = i1.reshape(T)
    i2 = i2.reshape(T)

    flat_e = jnp.concatenate([i1, i2])
    onehot = (flat_e[:, None] == jnp.arange(E, dtype=jnp.int32)[None, :])
    oh32 = onehot.astype(jnp.int32)
    csum = jnp.cumsum(oh32, axis=0)
    rank = jnp.sum(csum * oh32, axis=1) - 1
    counts = csum[-1]
    padded = ((counts + BLK - 1) // BLK) * BLK
    ends = jnp.cumsum(padded)
    offs = ends - padded
    off_per_a = jnp.sum(offs[None, :] * oh32, axis=1)
    dest = (off_per_a + rank).astype(jnp.int32)

    Rp = T * 2 + E * BLK
    NB = Rp // BLK
    CH = 32
    nch = (2 * T) // (_NW * CH)
    dest3 = dest.reshape(_NW, nch, CH)
    block_starts = jnp.arange(NB, dtype=jnp.int32) * BLK
    block_expert = jnp.sum(
        (block_starts[:, None] >= ends[None, :]).astype(jnp.int32), axis=1)
    block_expert = jnp.minimum(block_expert, E - 1).astype(jnp.int32)
    nb_active = (ends[-1] // BLK).astype(jnp.int32).reshape(1)

    be = block_expert
    is_first = jnp.concatenate(
        [jnp.ones((1,), jnp.bool_), be[1:] != be[:-1]])
    run_id = jnp.cumsum(is_first.astype(jnp.int32)) - 1
    slot_arr = (run_id % 2).astype(jnp.int32)
    iota_nb = jnp.arange(NB, dtype=jnp.int32)
    arr = jnp.where(is_first, iota_nb, NB)
    min_from = lax.cummin(arr[::-1])[::-1]
    nxt_t = jnp.concatenate([min_from[1:], jnp.full((1,), NB, jnp.int32)])
    nre_arr = be[jnp.minimum(nxt_t, NB - 1)].astype(jnp.int32)
    start_arr = (is_first & (nxt_t < nb_active[0])).astype(jnp.int32)

    xs = _dispatch(x, dest3, Rp)
    wp = _ffn(xs, W1, b1, W2, b2, block_expert, nb_active, slot_arr,
              start_arr, nre_arr)
    y = _combine(wp, dest[:T], dest[T:], g1sp, g2sp, T)

    eps = 1e-10
    imp = imp.reshape(E)
    load = load.reshape(E)
    cv_imp = jnp.var(imp, ddof=1) / (jnp.mean(imp) ** 2 + eps)
    cv_load = jnp.var(load, ddof=1) / (jnp.mean(load) ** 2 + eps)
    loss = (cv_imp + cv_load) * 0.01
    return (y, loss)
